# baseline (speedup 1.0000x reference)
import functools

import jax
import jax.numpy as jnp
from jax import lax
from jax.experimental import pallas as pl
from jax.experimental.pallas import tpu as pltpu


def _fd_body(c1_ref, c0_ref, xb_ref, rhs_ref, o_ref, *, dmodel):
    xb = xb_ref[...]
    rhs = rhs_ref[...]

    a = (jnp.sum(xb * xb, axis=1, keepdims=True)
         + 2e-6 * jnp.sum(xb, axis=1, keepdims=True))
    ones = jnp.ones_like(a)
    lhs = jnp.concatenate([xb * (-2.0), a, ones], axis=1)
    d2 = jnp.dot(lhs, rhs, preferred_element_type=jnp.float32,
                 precision=lax.Precision.HIGHEST)

    dist = jnp.sqrt(jnp.maximum(d2, 0.0))

    fs = [1.0 / (jnp.exp2(dist * c1_ref[k] + c0_ref[k]) + 1.0) for k in range(3)]
    noedge = 1.0 - jnp.maximum(fs[0], jnp.maximum(fs[1], fs[2]))
    planes = [noedge] + fs

    n = dist.shape[1]
    pieces = [p[:, jt * 128:(jt + 1) * 128]
              for jt in range(n // 128) for p in planes]
    res = jnp.concatenate(pieces, axis=1)
    o_ref[...] = res.reshape(o_ref.shape)


@jax.jit
def kernel(x, r, t):
    b, n, dmodel = x.shape
    nt = n // 128
    x2 = x[0]
    x2t = x2.T
    bj = (jnp.sum(x2t * x2t, axis=0, keepdims=True)
          - 2e-6 * jnp.sum(x2t, axis=0, keepdims=True)
          + dmodel * 1e-12)
    rhs = jnp.concatenate([x2t, jnp.ones_like(bj), bj], axis=0)
    log2e = 1.4426950408889634
    c1 = t * log2e
    c0 = -r * t * log2e

    bi = 128
    grid = (n // bi,)
    out = pl.pallas_call(
        functools.partial(_fd_body, dmodel=dmodel),
        grid=grid,
        in_specs=[
            pl.BlockSpec(memory_space=pltpu.SMEM),
            pl.BlockSpec(memory_space=pltpu.SMEM),
            pl.BlockSpec((bi, dmodel), lambda i: (i, 0)),
            pl.BlockSpec((dmodel + 2, n), lambda i: (0, 0)),
        ],
        out_specs=pl.BlockSpec((bi, 4 * nt, 128), lambda i: (i, 0, 0)),
        out_shape=jax.ShapeDtypeStruct((n, 4 * nt, 128), jnp.float32),
    )(c1, c0, x2, rhs)
    out = out.reshape(n, nt, 4, 128).transpose(0, 1, 3, 2)
    return out.reshape(b, n, n, 4)


# speedup vs baseline: 1.0059x; 1.0059x over previous
import functools

import jax
import jax.numpy as jnp
from jax import lax
from jax.experimental import pallas as pl
from jax.experimental.pallas import tpu as pltpu


def _fd_body(c1_ref, c0_ref, xb_ref, rhs_ref, o_ref, *, dmodel):
    xb = xb_ref[...]
    rhs = rhs_ref[...]

    x2t = rhs[:dmodel, :]
    bj = rhs[dmodel + 1:, :]
    dot = jnp.dot(xb * (-2.0), x2t, preferred_element_type=jnp.float32,
                  precision=lax.Precision.HIGHEST)
    a = (jnp.sum(xb * xb, axis=1, keepdims=True)
         + 2e-6 * jnp.sum(xb, axis=1, keepdims=True))
    dist = jnp.sqrt(jnp.maximum(dot + a + bj, 0.0))

    fs = [1.0 / (jnp.exp2(dist * c1_ref[k] + c0_ref[k]) + 1.0) for k in range(3)]
    noedge = 1.0 - jnp.maximum(fs[0], jnp.maximum(fs[1], fs[2]))
    planes = [noedge] + fs

    n = dist.shape[1]
    pieces = [p[:, jt * 128:(jt + 1) * 128]
              for jt in range(n // 128) for p in planes]
    res = jnp.concatenate(pieces, axis=1)
    o_ref[...] = res.reshape(o_ref.shape)


@jax.jit
def kernel(x, r, t):
    b, n, dmodel = x.shape
    nt = n // 128
    x2 = x[0]
    x2t = x2.T
    bj = (jnp.sum(x2t * x2t, axis=0, keepdims=True)
          - 2e-6 * jnp.sum(x2t, axis=0, keepdims=True)
          + dmodel * 1e-12)
    rhs = jnp.concatenate([x2t, jnp.ones_like(bj), bj], axis=0)
    log2e = 1.4426950408889634
    c1 = t * log2e
    c0 = -r * t * log2e

    bi = 128
    grid = (n // bi,)
    out = pl.pallas_call(
        functools.partial(_fd_body, dmodel=dmodel),
        grid=grid,
        in_specs=[
            pl.BlockSpec(memory_space=pltpu.SMEM),
            pl.BlockSpec(memory_space=pltpu.SMEM),
            pl.BlockSpec((bi, dmodel), lambda i: (i, 0)),
            pl.BlockSpec((dmodel + 2, n), lambda i: (0, 0)),
        ],
        out_specs=pl.BlockSpec((bi, 4 * nt, 128), lambda i: (i, 0, 0)),
        out_shape=jax.ShapeDtypeStruct((n, 4 * nt, 128), jnp.float32),
    )(c1, c0, x2, rhs)
    out = out.reshape(n, nt, 4, 128).transpose(0, 1, 3, 2)
    return out.reshape(b, n, n, 4)


# speedup vs baseline: 1.0908x; 1.0844x over previous
import functools

import jax
import jax.numpy as jnp
from jax import lax
from jax.experimental import pallas as pl
from jax.experimental.pallas import tpu as pltpu


def _fd_body(r_ref, t_ref, xb_ref, x2t_ref, o_ref, *, dmodel):
    xb = xb_ref[...]
    x2t = x2t_ref[...]

    dot = jnp.dot(xb * (-2.0), x2t, preferred_element_type=jnp.float32,
                  precision=lax.Precision.HIGHEST)

    a = (jnp.sum(xb * xb, axis=1, keepdims=True)
         + 2e-6 * jnp.sum(xb, axis=1, keepdims=True))
    bc = (jnp.sum(x2t * x2t, axis=0, keepdims=True)
          - 2e-6 * jnp.sum(x2t, axis=0, keepdims=True)
          + dmodel * 1e-12)

    dist = jnp.sqrt(jnp.maximum(dot + a + bc, 0.0))

    fs = [1.0 / (jnp.exp2(dist * r_ref[k] + t_ref[k]) + 1.0) for k in range(3)]
    noedge = 1.0 - jnp.maximum(fs[0], jnp.maximum(fs[1], fs[2]))
    planes = [noedge] + fs

    n = dist.shape[1]
    pieces = [p[:, jt * 128:(jt + 1) * 128]
              for jt in range(n // 128) for p in planes]
    res = jnp.concatenate(pieces, axis=1)
    o_ref[...] = res.reshape(o_ref.shape)


@jax.jit
def kernel(x, r, t):
    b, n, dmodel = x.shape
    nt = n // 128
    x2 = x[0]
    x2t = x2.T

    bi = 128
    grid = (n // bi,)
    out = pl.pallas_call(
        functools.partial(_fd_body, dmodel=dmodel),
        grid=grid,
        in_specs=[
            pl.BlockSpec(memory_space=pltpu.SMEM),
            pl.BlockSpec(memory_space=pltpu.SMEM),
            pl.BlockSpec((bi, dmodel), lambda i: (i, 0)),
            pl.BlockSpec((dmodel, n), lambda i: (0, 0)),
        ],
        out_specs=pl.BlockSpec((bi, 4 * nt, 128), lambda i: (i, 0, 0)),
        out_shape=jax.ShapeDtypeStruct((n, 4 * nt, 128), jnp.float32),
    )(t * 1.4426950408889634, -r * t * 1.4426950408889634, x2, x2t)
    out = out.reshape(n, nt, 4, 128).transpose(0, 1, 3, 2)
    return out.reshape(b, n, n, 4)


# speedup vs baseline: 1.2632x; 1.1581x over previous
import functools

import jax
import jax.numpy as jnp
from jax import lax
from jax.experimental import pallas as pl
from jax.experimental.pallas import tpu as pltpu


def _fd_body(r_ref, t_ref, xb_ref, x2t_ref, o_ref, *, dmodel):
    xb = xb_ref[...]
    x2t = x2t_ref[...]

    dot = jnp.dot(xb * (-2.0), x2t, preferred_element_type=jnp.float32)

    a = (jnp.sum(xb * xb, axis=1, keepdims=True)
         + 2e-6 * jnp.sum(xb, axis=1, keepdims=True))
    bc = (jnp.sum(x2t * x2t, axis=0, keepdims=True)
          - 2e-6 * jnp.sum(x2t, axis=0, keepdims=True)
          + dmodel * 1e-12)

    dist = jnp.sqrt(jnp.maximum(dot + a + bc, 0.0))

    fs = [1.0 / (jnp.exp2(dist * r_ref[k] + t_ref[k]) + 1.0) for k in range(3)]
    noedge = 1.0 - jnp.maximum(fs[0], jnp.maximum(fs[1], fs[2]))
    planes = [noedge] + fs

    n = dist.shape[1]
    pieces = [p[:, jt * 128:(jt + 1) * 128]
              for jt in range(n // 128) for p in planes]
    res = jnp.concatenate(pieces, axis=1)
    o_ref[...] = res.reshape(o_ref.shape)


@jax.jit
def kernel(x, r, t):
    b, n, dmodel = x.shape
    nt = n // 128
    x2 = x[0]
    x2t = x2.T

    bi = 128
    grid = (n // bi,)
    out = pl.pallas_call(
        functools.partial(_fd_body, dmodel=dmodel),
        grid=grid,
        in_specs=[
            pl.BlockSpec(memory_space=pltpu.SMEM),
            pl.BlockSpec(memory_space=pltpu.SMEM),
            pl.BlockSpec((bi, dmodel), lambda i: (i, 0)),
            pl.BlockSpec((dmodel, n), lambda i: (0, 0)),
        ],
        out_specs=pl.BlockSpec((bi, 4 * nt, 128), lambda i: (i, 0, 0)),
        out_shape=jax.ShapeDtypeStruct((n, 4 * nt, 128), jnp.float32),
    )(t * 1.4426950408889634, -r * t * 1.4426950408889634, x2, x2t)
    out = out.reshape(n, nt, 4, 128).transpose(0, 1, 3, 2)
    return out.reshape(b, n, n, 4)
